# Initial kernel scaffold; baseline (speedup 1.0000x reference)
import jax
import jax.numpy as jnp
from jax.experimental import pallas as pl


def kernel(x, edge_index, batch, W1a, b1a, W1b, b1b, W2a, b2a, W2b, b2b, Wf, bf):
    raise NotImplementedError("write your pallas kernel here")


# speedup vs baseline: 5.9820x; 5.9820x over previous
import functools

import jax
import jax.numpy as jnp
from jax import lax
from jax.experimental import pallas as pl
from jax.experimental.pallas import tpu as pltpu
from jax.experimental.pallas import tpu_sc as plsc

_NC = 2
_NS = 16
_LANES = 128
_G = 64
_NBUF = 4


@functools.cache
def _segsum_call(n, h, e_pad):
    ntiles = _NC * _NS
    ept = e_pad // ntiles
    blocks = ept // _LANES
    assert blocks * _LANES * ntiles == e_pad
    assert blocks % _NBUF == 0
    assert n % 8 == 0
    zr = (n // _NS + 8) // 8 * 8
    n_acc = zr * _NS
    assert n_acc > n
    wr_last = n - zr * (_NS - 1)
    assert wr_last > 0 and wr_last % 8 == 0

    mesh = plsc.VectorSubcoreMesh(core_axis_name="c", subcore_axis_name="s")

    @functools.partial(
        pl.kernel,
        out_type=jax.ShapeDtypeStruct((_NC * n, h), jnp.float32),
        mesh=mesh,
        scratch_types=[
            pltpu.VMEM((blocks, _LANES), jnp.int32),
            pltpu.VMEM((blocks, _LANES), jnp.int32),
            pltpu.VMEM((_NBUF, _LANES, h), jnp.float32),
            pltpu.VMEM_SHARED((n_acc, h), jnp.float32),
        ] + [pltpu.SemaphoreType.DMA] * _NBUF,
        compiler_params=pltpu.CompilerParams(use_tc_tiling_on_sc=False),
    )
    def segsum(t_hbm, src_hbm, dst_hbm, zero_hbm, out_hbm,
               sidx, didx, rows, acc, *sems):
        c = lax.axis_index("c")
        s = lax.axis_index("s")
        wid = s * _NC + c
        pltpu.sync_copy(zero_hbm.at[pl.ds(s * zr, zr)],
                        acc.at[pl.ds(s * zr, zr)])
        pltpu.sync_copy(src_hbm.at[pl.ds(wid * blocks, blocks)], sidx)
        pltpu.sync_copy(dst_hbm.at[pl.ds(wid * blocks, blocks)], didx)
        plsc.subcore_barrier()

        for b in range(_NBUF):
            pltpu.make_async_copy(t_hbm.at[sidx.at[b]], rows.at[b],
                                  sems[b]).start()

        def body(k, carry):
            j0 = _NBUF * k
            for b in range(_NBUF):
                j = j0 + b
                pltpu.make_async_copy(t_hbm.at[sidx.at[j]], rows.at[b],
                                      sems[b]).wait()
                pltpu.sync_copy(rows.at[b], acc.at[didx.at[j]], add=True)
                jn = lax.rem(j + _NBUF, blocks)
                pltpu.make_async_copy(t_hbm.at[sidx.at[jn]], rows.at[b],
                                      sems[b]).start()
            return carry

        lax.fori_loop(0, blocks // _NBUF, body, 0)
        for b in range(_NBUF):
            pltpu.make_async_copy(t_hbm.at[sidx.at[b]], rows.at[b],
                                  sems[b]).wait()
        plsc.subcore_barrier()

        @pl.when(s < _NS - 1)
        def _():
            pltpu.sync_copy(acc.at[pl.ds(s * zr, zr)],
                            out_hbm.at[pl.ds(c * n + s * zr, zr)])

        @pl.when(s == _NS - 1)
        def _():
            pltpu.sync_copy(acc.at[pl.ds((_NS - 1) * zr, wr_last)],
                            out_hbm.at[pl.ds(c * n + (_NS - 1) * zr, wr_last)])

    return segsum


def _mm_body(x_ref, w_ref, o_ref):
    o_ref[...] = jnp.dot(x_ref[...], w_ref[...],
                         preferred_element_type=jnp.float32)


def _mlp_body(y_ref, p_ref, ba_ref, wb_ref, bb_ref, h_ref):
    n = y_ref.shape[0]
    p = p_ref[...]
    t = y_ref[...] + p[:n] + p[n:] + ba_ref[...]
    t = jnp.maximum(t, 0.0)
    t = jnp.dot(t, wb_ref[...], preferred_element_type=jnp.float32) + bb_ref[...]
    h_ref[...] = jnp.maximum(t, 0.0)


def _mlp2_pool_body(h_ref, p_ref, w2a_ref, b2a_ref, w2b_ref, b2b_ref,
                    batch_ref, wf_ref, bf_ref, o_ref):
    n = h_ref.shape[0]
    p = p_ref[...]
    t = h_ref[...] + p[:n] + p[n:]
    t = jnp.dot(t, w2a_ref[...], preferred_element_type=jnp.float32) + b2a_ref[...]
    t = jnp.maximum(t, 0.0)
    t = jnp.dot(t, w2b_ref[...], preferred_element_type=jnp.float32) + b2b_ref[...]
    h2 = jnp.maximum(t, 0.0)
    gids = lax.broadcasted_iota(jnp.int32, (_G, 1), 0)
    mask = (batch_ref[...] == gids).astype(jnp.float32)
    sums = jnp.dot(mask, h2, preferred_element_type=jnp.float32)
    counts = jnp.sum(mask, axis=1, keepdims=True)
    pooled = sums / jnp.maximum(counts, 1.0)
    o_ref[...] = jnp.dot(pooled, wf_ref[...],
                         preferred_element_type=jnp.float32) + bf_ref[...]


def kernel(x, edge_index, batch, W1a, b1a, W1b, b1b, W2a, b2a, W2b, b2b,
           Wf, bf):
    n, d = x.shape
    h = W1a.shape[1]
    e = edge_index.shape[1]

    chunk = _NC * _NS * _LANES * _NBUF
    e_pad = ((e + chunk - 1) // chunk) * chunk
    src = edge_index[0].astype(jnp.int32)
    dst = edge_index[1].astype(jnp.int32)
    pad = e_pad - e
    src_p = jnp.concatenate([src, jnp.zeros((pad,), jnp.int32)]
                            ).reshape(-1, _LANES)
    dst_p = jnp.concatenate([dst, jnp.full((pad,), n, jnp.int32)]
                            ).reshape(-1, _LANES)
    zr = (n // _NS + 8) // 8 * 8
    zeros_rows = jnp.zeros((zr * _NS, h), jnp.float32)
    batch2d = batch.astype(jnp.int32).reshape(1, n)

    segsum = _segsum_call(n, h, e_pad)

    y = pl.pallas_call(
        _mm_body,
        out_shape=jax.ShapeDtypeStruct((n, h), jnp.float32),
    )(x, W1a)
    p1 = segsum(y, src_p, dst_p, zeros_rows)
    h1 = pl.pallas_call(
        _mlp_body,
        out_shape=jax.ShapeDtypeStruct((n, h), jnp.float32),
    )(y, p1, b1a.reshape(1, h), W1b, b1b.reshape(1, h))

    p2 = segsum(h1, src_p, dst_p, zeros_rows)
    out = pl.pallas_call(
        _mlp2_pool_body,
        out_shape=jax.ShapeDtypeStruct((_G, 1), jnp.float32),
    )(h1, p2, W2a, b2a.reshape(1, h), W2b, b2b.reshape(1, h),
      batch2d, Wf, bf.reshape(1, 1))
    return out[:, 0]
